# Initial kernel scaffold; baseline (speedup 1.0000x reference)
import jax
import jax.numpy as jnp
from jax.experimental import pallas as pl


def kernel(pcs):
    raise NotImplementedError("write your pallas kernel here")


# speedup vs baseline: 20.5076x; 20.5076x over previous
import jax
import jax.numpy as jnp
from jax.experimental import pallas as pl
from jax.experimental.pallas import tpu as pltpu

_K = 10
_N_SEEDS = 20


def _knn_body(xt_ref, f0_ref, out_ref, s11_ref, d0_ref):
    B, N = xt_ref.shape[1], xt_ref.shape[2]
    x = xt_ref[0]
    y = xt_ref[1]
    z = xt_ref[2]
    iota_n = jax.lax.broadcasted_iota(jnp.int32, (B, N), 1)

    far = f0_ref[:, :]
    distance = jnp.full((B, N), 1e10, dtype=jnp.float32)

    for s in range(_N_SEEDS):
        onehot = iota_n == far
        cx = jnp.sum(jnp.where(onehot, x, 0.0), axis=1, keepdims=True)
        cy = jnp.sum(jnp.where(onehot, y, 0.0), axis=1, keepdims=True)
        cz = jnp.sum(jnp.where(onehot, z, 0.0), axis=1, keepdims=True)
        dx = x - cx
        dy = y - cy
        dz = z - cz
        d2 = (dx * dx + dy * dy) + dz * dz

        distance = jnp.minimum(distance, d2)
        m = jnp.max(distance, axis=1, keepdims=True)
        far = jnp.min(
            jnp.where(distance == m, iota_n, N), axis=1, keepdims=True
        )

        work = d2
        need = jnp.full((B, 1), float(_K + 1), dtype=jnp.float32)
        acc = jnp.zeros((B, 1), dtype=jnp.float32)
        for j in range(_K + 1):
            mv = jnp.min(work, axis=1, keepdims=True)
            eq = work == mv
            cnt = jnp.sum(jnp.where(eq, 1.0, 0.0), axis=1, keepdims=True)
            take = jnp.minimum(cnt, need)
            rt = jnp.sqrt(mv)
            if j == 0:
                d0_ref[:, s : s + 1] = rt
            acc = acc + take * rt
            need = need - take
            work = jnp.where(eq, jnp.float32(jnp.inf), work)
        s11_ref[:, s : s + 1] = acc

    s11 = s11_ref[:, :]
    d0 = d0_ref[:, :]
    n_rows = B * _N_SEEDS
    overall_mean = (jnp.sum(s11) - jnp.sum(d0)) / float(n_rows * _K)
    mrow = s11 / (float(_K + 1) * overall_mean)
    mean_m = jnp.sum(mrow) / float(n_rows)
    dev = mrow - mean_m
    out_ref[0, 0] = jnp.sum(dev * dev) / float(n_rows - 1)


def kernel(pcs):
    B, N, C = pcs.shape
    xt = jnp.transpose(pcs, (2, 0, 1))
    f0 = jax.random.randint(jax.random.key(1), (B,), 0, N)
    f0 = f0.astype(jnp.int32).reshape(B, 1)

    out = pl.pallas_call(
        _knn_body,
        out_shape=jax.ShapeDtypeStruct((1, 1), jnp.float32),
        out_specs=pl.BlockSpec(memory_space=pltpu.SMEM),
        scratch_shapes=[
            pltpu.VMEM((B, _N_SEEDS), jnp.float32),
            pltpu.VMEM((B, _N_SEEDS), jnp.float32),
        ],
    )(xt, f0)
    return out[0, 0]
